# Initial kernel scaffold; baseline (speedup 1.0000x reference)
import jax
import jax.numpy as jnp
from jax.experimental import pallas as pl


def kernel(vlabels, edge_index, Wl2c0, bl2c0, Wc2l0, bc2l0, lng0, lnb0, Wl2c1, bl2c1, Wc2l1, bc2l1, lng1, lnb1, Wl2c2, bl2c2, Wc2l2, bc2l2, lng2, lnb2):
    raise NotImplementedError("write your pallas kernel here")


# speedup vs baseline: 4.2791x; 4.2791x over previous
import functools

import jax
import jax.numpy as jnp
from jax import lax
from jax.experimental import pallas as pl
from jax.experimental.pallas import tpu as pltpu
from jax.experimental.pallas import tpu_sc as plsc

NL = 10000
NC = 10000
E = 320000
D = 128

NCORES = 2
NSUB = 16
NW = NCORES * NSUB
EPW = E // NW
CHUNK = 80
NCHUNK = EPW // CHUNK

SEG_PAD = 10240
ROWS_PT = SEG_PAD // NSUB


def _seg_sum_partials(vals, gidx, sidx, zeros_hbm):
    mesh = plsc.VectorSubcoreMesh(core_axis_name="c", subcore_axis_name="s")

    @functools.partial(
        pl.kernel,
        out_type=jax.ShapeDtypeStruct((NCORES, SEG_PAD, D), jnp.float32),
        mesh=mesh,
        scratch_types=[
            pltpu.VMEM_SHARED((SEG_PAD, D), jnp.float32),
            pltpu.VMEM((CHUNK,), jnp.int32),
            pltpu.VMEM((CHUNK,), jnp.int32),
            pltpu.VMEM((CHUNK, D), jnp.float32),
            pltpu.SemaphoreType.DMA,
        ],
    )
    def k(vals_hbm, gidx_hbm, sidx_hbm, z_hbm, out_hbm, acc_sh, gi_v, si_v,
          rows_v, sem):
        c = lax.axis_index("c")
        s = lax.axis_index("s")
        wid = c * NSUB + s

        pltpu.sync_copy(z_hbm, acc_sh.at[pl.ds(s * ROWS_PT, ROWS_PT)])
        plsc.subcore_barrier()

        def body(j, carry):
            base = wid * EPW + j * CHUNK
            pltpu.sync_copy(gidx_hbm.at[pl.ds(base, CHUNK)], gi_v)
            pltpu.sync_copy(sidx_hbm.at[pl.ds(base, CHUNK)], si_v)
            pltpu.async_copy(vals_hbm.at[gi_v], rows_v, sem).wait()
            pltpu.sync_copy(rows_v, acc_sh.at[si_v], add=True)
            return carry

        lax.fori_loop(0, NCHUNK, body, 0)
        plsc.subcore_barrier()

        pltpu.sync_copy(acc_sh.at[pl.ds(s * ROWS_PT, ROWS_PT)],
                        out_hbm.at[c, pl.ds(s * ROWS_PT, ROWS_PT)])

    return k(vals, gidx, sidx, zeros_hbm)


BR = 2000


def _dense_in(x, W, b):
    n, kdim = x.shape

    def body(x_ref, w_ref, b_ref, o_ref):
        o_ref[...] = (
            jnp.dot(x_ref[...], w_ref[...], preferred_element_type=jnp.float32)
            + b_ref[...])

    return pl.pallas_call(
        body,
        grid=(n // BR,),
        in_specs=[
            pl.BlockSpec((BR, kdim), lambda i: (i, 0)),
            pl.BlockSpec((kdim, D), lambda i: (0, 0)),
            pl.BlockSpec((1, D), lambda i: (0, 0)),
        ],
        out_specs=pl.BlockSpec((BR, D), lambda i: (i, 0)),
        out_shape=jax.ShapeDtypeStruct((n, D), jnp.float32),
    )(x, W, b)


def _dense_mid(cpart, W, b):
    def body(cp_ref, w_ref, b_ref, ce_ref, m2_ref):
        ce = jnp.maximum(cp_ref[0] + cp_ref[1], 0.0)
        ce_ref[...] = ce
        m2_ref[...] = (
            jnp.dot(ce, w_ref[...], preferred_element_type=jnp.float32)
            + b_ref[...])

    return pl.pallas_call(
        body,
        grid=(NC // BR,),
        in_specs=[
            pl.BlockSpec((2, BR, D), lambda i: (0, i, 0)),
            pl.BlockSpec((D, D), lambda i: (0, 0)),
            pl.BlockSpec((1, D), lambda i: (0, 0)),
        ],
        out_specs=[
            pl.BlockSpec((BR, D), lambda i: (i, 0)),
            pl.BlockSpec((BR, D), lambda i: (i, 0)),
        ],
        out_shape=[
            jax.ShapeDtypeStruct((NC, D), jnp.float32),
            jax.ShapeDtypeStruct((NC, D), jnp.float32),
        ],
    )(cpart, W, b)


def _dense_out(lpart, g, b):
    def body(lp_ref, g_ref, b_ref, o_ref):
        x = jnp.maximum(lp_ref[0] + lp_ref[1], 0.0)
        mu = jnp.mean(x, axis=-1, keepdims=True)
        var = jnp.mean((x - mu) ** 2, axis=-1, keepdims=True)
        o_ref[...] = (x - mu) * lax.rsqrt(var + 1e-5) * g_ref[...] + b_ref[...]

    return pl.pallas_call(
        body,
        grid=(NL // BR,),
        in_specs=[
            pl.BlockSpec((2, BR, D), lambda i: (0, i, 0)),
            pl.BlockSpec((1, D), lambda i: (0, 0)),
            pl.BlockSpec((1, D), lambda i: (0, 0)),
        ],
        out_specs=pl.BlockSpec((BR, D), lambda i: (i, 0)),
        out_shape=jax.ShapeDtypeStruct((NL, D), jnp.float32),
    )(lpart, g, b)


def _tie(pre):
    z = pre.reshape(-1, 2, D)
    rc = z[:, ::-1, :].reshape(-1, D)
    return jnp.concatenate([pre, rc], axis=1)


def kernel(vlabels, edge_index, Wl2c0, bl2c0, Wc2l0, bc2l0, lng0, lnb0,
           Wl2c1, bl2c1, Wc2l1, bc2l1, lng1, lnb1,
           Wl2c2, bl2c2, Wc2l2, bc2l2, lng2, lnb2):
    src = edge_index[0]
    dst = edge_index[1]
    zeros_hbm = jnp.zeros((ROWS_PT, D), jnp.float32)
    params = [
        (Wl2c0, bl2c0, Wc2l0, bc2l0, lng0, lnb0),
        (Wl2c1, bl2c1, Wc2l1, bc2l1, lng1, lnb1),
        (Wl2c2, bl2c2, Wc2l2, bc2l2, lng2, lnb2),
    ]
    h = vlabels
    cembs = None
    for (Wa, ba, Wb, bb, g, b) in params:
        m = _dense_in(h, Wa, ba.reshape(1, D))
        cpart = _seg_sum_partials(m, src, dst, zeros_hbm)
        cembs, m2 = _dense_mid(cpart, Wb, bb.reshape(1, D))
        lpart = _seg_sum_partials(m2, dst, src, zeros_hbm)
        pre = _dense_out(lpart, g.reshape(1, D), b.reshape(1, D))
        h = _tie(pre)
    return (h, cembs)
